# baseline (speedup 1.0000x reference)
import functools

import jax
import jax.numpy as jnp
from jax import lax
from jax.experimental import pallas as pl
from jax.experimental.pallas import tpu as pltpu
from jax.experimental.pallas import tpu_sc as plsc

DEPTH = 1000
ON_VALUE = 1.0
OFF_VALUE = 0.0

NUM_CORES = 2
NUM_SUBCORES = 16
NUM_WORKERS = NUM_CORES * NUM_SUBCORES
LANES = 16

CHUNK_B = 2
F_PAD = 32
D_PAD = 1024
BATCH_WORDS = F_PAD * D_PAD

SC_BATCHES = 256
TC_BLOCK_B = 16


def _one_hot_sc_image(idx_flat, f_total):
  batches_per_worker = SC_BATCHES // NUM_WORKERS
  n_chunks = batches_per_worker // CHUNK_B
  chunk_rows = CHUNK_B * f_total
  rows_per_worker = batches_per_worker * f_total
  n_groups = -(-chunk_rows // LANES)

  mesh = plsc.VectorSubcoreMesh(core_axis_name="c", subcore_axis_name="s")

  @functools.partial(
      pl.kernel,
      mesh=mesh,
      out_type=jax.ShapeDtypeStruct((SC_BATCHES * BATCH_WORDS,), jnp.float32),
      scratch_types=[
          pltpu.VMEM((rows_per_worker,), jnp.int32),
          pltpu.VMEM((CHUNK_B * BATCH_WORDS,), jnp.float32),
      ],
      compiler_params=pltpu.CompilerParams(needs_layout_passes=False),
  )
  def k(idx_hbm, out_hbm, idx_v, buf):
    wid = lax.axis_index("s") * NUM_CORES + lax.axis_index("c")
    batch0 = wid * batches_per_worker

    pltpu.sync_copy(idx_hbm.at[pl.ds(batch0 * f_total, rows_per_worker)],
                    idx_v)

    zeros16 = jnp.zeros((LANES,), jnp.float32)

    def zero_body(i, _):
      base = i * (8 * LANES)
      for u in range(8):
        buf[pl.ds(base + u * LANES, LANES)] = zeros16
      return 0

    lax.fori_loop(0, CHUNK_B * BATCH_WORDS // (8 * LANES), zero_body, 0)

    lane = lax.iota(jnp.int32, LANES)
    ones16 = jnp.full((LANES,), jnp.float32(ON_VALUE))

    def scatter_chunk(c, val16):
      for g in range(n_groups):
        j = lane + g * LANES
        mask = j < chunk_rows if (g + 1) * LANES > chunk_rows else None
        d = plsc.load_gather(idx_v, [j + c * chunk_rows], mask=mask)
        b = jnp.where(j >= f_total, 1, 0)
        f = j - b * f_total
        off = (b * BATCH_WORDS + (f >> 3) * (8 * D_PAD) + (d >> 7) * 1024
               + (f & 7) * 128 + (d & 127))
        plsc.store_scatter(buf, [off], val16, mask=mask)

    def chunk_body(c, _):
      scatter_chunk(c, ones16)
      pltpu.sync_copy(
          buf,
          out_hbm.at[pl.ds((batch0 + c * CHUNK_B) * BATCH_WORDS,
                           CHUNK_B * BATCH_WORDS)])
      scatter_chunk(c, zeros16)
      return 0

    lax.fori_loop(0, n_chunks, chunk_body, 0)

  return k(idx_flat)


def _tc_body(sc_blocks, img_ref, idx_ref, out_ref, scratch, sem):
  i = pl.program_id(0)

  @pl.when(i < sc_blocks)
  def _assemble():
    rows_per_block = TC_BLOCK_B * 256
    cp = pltpu.make_async_copy(
        img_ref.at[pl.ds(i * rows_per_block, rows_per_block)], scratch, sem)
    cp.start()
    cp.wait()
    for bb in range(TC_BLOCK_B):
      for g in range(4):
        rows = 8 if g < 3 else 2
        for c0 in range(8):
          cols = 128 if c0 < 7 else DEPTH - 7 * 128
          out_ref[bb, pl.ds(8 * g, rows), pl.ds(128 * c0, cols)] = (
              scratch[pl.ds(bb * 256 + 8 * (8 * g + c0), rows),
                      pl.ds(0, cols)])

  @pl.when(i >= sc_blocks)
  def _compute():
    iota_d = lax.broadcasted_iota(jnp.int32, (TC_BLOCK_B, 26, DEPTH), 2)
    idx_b = idx_ref[...][:, :, None]
    out_ref[...] = jnp.where(idx_b == iota_d, jnp.float32(ON_VALUE),
                             jnp.float32(OFF_VALUE))


@jax.jit
def kernel(indices):
  b_total, f_total = indices.shape
  n_blocks = b_total // TC_BLOCK_B
  sc_blocks = SC_BATCHES // TC_BLOCK_B

  img = _one_hot_sc_image(indices.reshape(-1), f_total)
  img2d = img.reshape(-1, 128)

  out = pl.pallas_call(
      functools.partial(_tc_body, sc_blocks),
      grid=(n_blocks,),
      in_specs=[
          pl.BlockSpec(memory_space=pl.ANY),
          pl.BlockSpec((TC_BLOCK_B, f_total), lambda i: (i, 0)),
      ],
      out_specs=pl.BlockSpec((TC_BLOCK_B, f_total, DEPTH),
                             lambda i: (i, 0, 0)),
      out_shape=jax.ShapeDtypeStruct((b_total, f_total, DEPTH), jnp.float32),
      scratch_shapes=[pltpu.VMEM((TC_BLOCK_B * 256, 128), jnp.float32),
                      pltpu.SemaphoreType.DMA],
  )(img2d, indices)
  return out


# speedup vs baseline: 2.3776x; 2.3776x over previous
import functools

import jax
import jax.numpy as jnp
from jax import lax
from jax.experimental import pallas as pl
from jax.experimental.pallas import tpu as pltpu
from jax.experimental.pallas import tpu_sc as plsc

DEPTH = 1000
ON_VALUE = 1.0
OFF_VALUE = 0.0

NUM_CORES = 2
NUM_SUBCORES = 16
NUM_WORKERS = NUM_CORES * NUM_SUBCORES
LANES = 16

B_TOTAL = 1024
D_SC = 200
D_BLK = 40
PLANE = D_SC * B_TOTAL
SC_CHUNK = PLANE // 2
SEG = D_BLK * B_TOTAL


def _one_hot_sc_image(idx_flat, f_total):
  n_chunks = PLANE // SC_CHUNK
  n_groups = B_TOTAL // LANES

  mesh = plsc.VectorSubcoreMesh(core_axis_name="c", subcore_axis_name="s")

  @functools.partial(
      pl.kernel,
      mesh=mesh,
      out_type=jax.ShapeDtypeStruct((f_total * PLANE,), jnp.float32),
      scratch_types=[
          pltpu.VMEM((f_total * B_TOTAL,), jnp.int32),
          pltpu.VMEM((SC_CHUNK,), jnp.float32),
      ],
      compiler_params=pltpu.CompilerParams(needs_layout_passes=False),
  )
  def k(idx_hbm, out_hbm, idx_v, buf):
    wid = lax.axis_index("s") * NUM_CORES + lax.axis_index("c")

    @pl.when(wid < f_total)
    def _worker():
      f = wid
      pltpu.sync_copy(idx_hbm, idx_v)

      zeros16 = jnp.zeros((LANES,), jnp.float32)

      def zero_body(i, _):
        base = i * (8 * LANES)
        for u in range(8):
          buf[pl.ds(base + u * LANES, LANES)] = zeros16
        return 0

      lax.fori_loop(0, SC_CHUNK // (8 * LANES), zero_body, 0)

      lane = lax.iota(jnp.int32, LANES)
      ones16 = jnp.full((LANES,), jnp.float32(ON_VALUE))

      def scatter_chunk(c, val16):
        lo = c * SC_CHUNK
        for g in range(n_groups):
          b = lane + g * LANES
          d = plsc.load_gather(idx_v, [b * f_total + f])
          off = ((d >> 3) * (8 * B_TOTAL) + (b >> 7) * 1024
                 + (d & 7) * 128 + (b & 127))
          ok = (d < D_SC) & (off >= lo) & (off < lo + SC_CHUNK)
          plsc.store_scatter(buf, [off - lo], val16, mask=ok)

      def chunk_body(c, _):
        scatter_chunk(c, ones16)
        pltpu.sync_copy(buf,
                        out_hbm.at[pl.ds(f * PLANE + c * SC_CHUNK, SC_CHUNK)])
        scatter_chunk(c, zeros16)
        return 0

      lax.fori_loop(0, n_chunks, chunk_body, 0)

  return k(idx_flat)


def _tc_body(f_total, img_ref, idxt_ref, out_ref, scratch, sem):
  i = pl.program_id(0)
  sc_blocks = D_SC // D_BLK

  @pl.when(i < sc_blocks)
  def _assemble():
    cps = []
    for f in range(f_total):
      cps.append(pltpu.make_async_copy(
          img_ref.at[pl.ds(f * PLANE + i * SEG, SEG)], scratch.at[f], sem))
      cps[-1].start()
    for cp in cps:
      cp.wait()
    for f in range(f_total):
      for trl in range(D_BLK // 8):
        for c0 in range(B_TOTAL // 128):
          vals = scratch[f, pl.ds(trl * (8 * B_TOTAL) + c0 * 1024, 1024)]
          out_ref[f, pl.ds(8 * trl, 8), pl.ds(128 * c0, 128)] = (
              vals.reshape(8, 128))

  @pl.when(i >= sc_blocks)
  def _compute():
    d = lax.broadcasted_iota(jnp.int32, (f_total, D_BLK, B_TOTAL), 1)
    d = d + i * D_BLK
    idx_b = idxt_ref[...][:, None, :]
    out_ref[...] = jnp.where(idx_b == d, jnp.float32(ON_VALUE),
                             jnp.float32(OFF_VALUE))


@jax.jit
def kernel(indices):
  b_total, f_total = indices.shape
  img = _one_hot_sc_image(indices.reshape(-1), f_total)
  idx_t = indices.T

  out_t = pl.pallas_call(
      functools.partial(_tc_body, f_total),
      grid=(DEPTH // D_BLK,),
      in_specs=[
          pl.BlockSpec(memory_space=pl.ANY),
          pl.BlockSpec((f_total, b_total), lambda i: (0, 0)),
      ],
      out_specs=pl.BlockSpec((f_total, D_BLK, b_total), lambda i: (0, i, 0)),
      out_shape=jax.ShapeDtypeStruct((f_total, DEPTH, b_total), jnp.float32),
      scratch_shapes=[pltpu.VMEM((f_total, SEG), jnp.float32),
                      pltpu.SemaphoreType.DMA],
  )(img, idx_t)
  return jnp.transpose(out_t, (2, 0, 1))


# speedup vs baseline: 2.4283x; 1.0213x over previous
import functools

import jax
import jax.numpy as jnp
from jax import lax
from jax.experimental import pallas as pl
from jax.experimental.pallas import tpu as pltpu
from jax.experimental.pallas import tpu_sc as plsc

DEPTH = 1000
ON_VALUE = 1.0
OFF_VALUE = 0.0

NUM_CORES = 2
NUM_SUBCORES = 16
NUM_WORKERS = NUM_CORES * NUM_SUBCORES
LANES = 16

B_TOTAL = 1024
D_SC = 200
D_BLK = 40
PLANE = D_SC * B_TOTAL
SC_CHUNK = PLANE // 2
SEG = D_BLK * B_TOTAL


def _one_hot_sc_image(idx_flat, f_total):
  n_chunks = PLANE // SC_CHUNK
  n_groups = B_TOTAL // LANES

  mesh = plsc.VectorSubcoreMesh(core_axis_name="c", subcore_axis_name="s")

  @functools.partial(
      pl.kernel,
      mesh=mesh,
      out_type=jax.ShapeDtypeStruct((f_total * PLANE,), jnp.float32),
      scratch_types=[
          pltpu.VMEM((f_total * B_TOTAL,), jnp.int32),
          pltpu.VMEM((SC_CHUNK,), jnp.float32),
      ],
      compiler_params=pltpu.CompilerParams(needs_layout_passes=False),
  )
  def k(idx_hbm, out_hbm, idx_v, buf):
    wid = lax.axis_index("s") * NUM_CORES + lax.axis_index("c")

    @pl.when(wid < f_total)
    def _worker():
      f = wid
      pltpu.sync_copy(idx_hbm, idx_v)

      zeros16 = jnp.zeros((LANES,), jnp.float32)

      def zero_body(i, _):
        base = i * (8 * LANES)
        for u in range(8):
          buf[pl.ds(base + u * LANES, LANES)] = zeros16
        return 0

      lax.fori_loop(0, SC_CHUNK // (8 * LANES), zero_body, 0)

      lane = lax.iota(jnp.int32, LANES)
      ones16 = jnp.full((LANES,), jnp.float32(ON_VALUE))

      def scatter_chunk(c, val16):
        lo = c * SC_CHUNK
        for g in range(n_groups):
          b = lane + g * LANES
          d = plsc.load_gather(idx_v, [b * f_total + f])
          off = ((d >> 3) * (8 * B_TOTAL) + (b >> 7) * 1024
                 + (d & 7) * 128 + (b & 127))
          ok = (d < D_SC) & (off >= lo) & (off < lo + SC_CHUNK)
          plsc.store_scatter(buf, [off - lo], val16, mask=ok)

      def chunk_body(c, _):
        scatter_chunk(c, ones16)
        pltpu.sync_copy(buf,
                        out_hbm.at[pl.ds(f * PLANE + c * SC_CHUNK, SC_CHUNK)])
        scatter_chunk(c, zeros16)
        return 0

      lax.fori_loop(0, n_chunks, chunk_body, 0)

  return k(idx_flat)


def _tc_compute_body(f_total, sc_blocks, idxt_ref, out_ref):
  i = pl.program_id(0)
  d = lax.broadcasted_iota(jnp.int32, (f_total, D_BLK, B_TOTAL), 1)
  d = d + (i + sc_blocks) * D_BLK
  idx_b = idxt_ref[...][:, None, :]
  out_ref[...] = jnp.where(idx_b == d, jnp.float32(ON_VALUE),
                           jnp.float32(OFF_VALUE))


def _tc_assemble_body(f_total, img_ref, part_ref, out_ref, scratch, sem):
  del part_ref
  i = pl.program_id(0)
  cps = []
  for f in range(f_total):
    cps.append(pltpu.make_async_copy(
        img_ref.at[pl.ds(f * PLANE + i * SEG, SEG)], scratch.at[f], sem))
    cps[-1].start()
  for cp in cps:
    cp.wait()
  for f in range(f_total):
    for trl in range(D_BLK // 8):
      for c0 in range(B_TOTAL // 128):
        vals = scratch[f, pl.ds(trl * (8 * B_TOTAL) + c0 * 1024, 1024)]
        out_ref[f, pl.ds(8 * trl, 8), pl.ds(128 * c0, 128)] = (
            vals.reshape(8, 128))


@jax.jit
def kernel(indices):
  b_total, f_total = indices.shape
  sc_blocks = D_SC // D_BLK
  img = _one_hot_sc_image(indices.reshape(-1), f_total)
  idx_t = indices.T

  part = pl.pallas_call(
      functools.partial(_tc_compute_body, f_total, sc_blocks),
      grid=(DEPTH // D_BLK - sc_blocks,),
      in_specs=[pl.BlockSpec((f_total, b_total), lambda i: (0, 0))],
      out_specs=pl.BlockSpec((f_total, D_BLK, b_total),
                             lambda i: (0, i + sc_blocks, 0)),
      out_shape=jax.ShapeDtypeStruct((f_total, DEPTH, b_total), jnp.float32),
  )(idx_t)

  out_t = pl.pallas_call(
      functools.partial(_tc_assemble_body, f_total),
      grid=(sc_blocks,),
      in_specs=[
          pl.BlockSpec(memory_space=pl.ANY),
          pl.BlockSpec(memory_space=pl.ANY),
      ],
      out_specs=pl.BlockSpec((f_total, D_BLK, b_total), lambda i: (0, i, 0)),
      out_shape=jax.ShapeDtypeStruct((f_total, DEPTH, b_total), jnp.float32),
      scratch_shapes=[pltpu.VMEM((f_total, SEG), jnp.float32),
                      pltpu.SemaphoreType.DMA],
      input_output_aliases={1: 0},
  )(img, part)
  return jnp.transpose(out_t, (2, 0, 1))


# speedup vs baseline: 2.8731x; 1.1832x over previous
import functools

import jax
import jax.numpy as jnp
from jax import lax
from jax.experimental import pallas as pl
from jax.experimental.pallas import tpu as pltpu
from jax.experimental.pallas import tpu_sc as plsc

DEPTH = 1000
ON_VALUE = 1.0
OFF_VALUE = 0.0

NUM_CORES = 2
NUM_SUBCORES = 16
NUM_WORKERS = NUM_CORES * NUM_SUBCORES
LANES = 16

B_TOTAL = 1024
D_SC = 120
D_BLK = 40
PLANE = D_SC * B_TOTAL
SC_CHUNK = PLANE // 2
SEG = D_BLK * B_TOTAL


def _one_hot_sc_image(idx_flat, f_total):
  n_chunks = PLANE // SC_CHUNK
  n_groups = B_TOTAL // LANES

  mesh = plsc.VectorSubcoreMesh(core_axis_name="c", subcore_axis_name="s")

  @functools.partial(
      pl.kernel,
      mesh=mesh,
      out_type=jax.ShapeDtypeStruct((f_total * PLANE,), jnp.float32),
      scratch_types=[
          pltpu.VMEM((f_total * B_TOTAL,), jnp.int32),
          pltpu.VMEM((SC_CHUNK,), jnp.float32),
      ],
      compiler_params=pltpu.CompilerParams(needs_layout_passes=False),
  )
  def k(idx_hbm, out_hbm, idx_v, buf):
    wid = lax.axis_index("s") * NUM_CORES + lax.axis_index("c")

    @pl.when(wid < f_total)
    def _worker():
      f = wid
      pltpu.sync_copy(idx_hbm, idx_v)

      zeros16 = jnp.zeros((LANES,), jnp.float32)

      def zero_body(i, _):
        base = i * (8 * LANES)
        for u in range(8):
          buf[pl.ds(base + u * LANES, LANES)] = zeros16
        return 0

      lax.fori_loop(0, SC_CHUNK // (8 * LANES), zero_body, 0)

      lane = lax.iota(jnp.int32, LANES)
      ones16 = jnp.full((LANES,), jnp.float32(ON_VALUE))

      def scatter_chunk(c, val16):
        lo = c * SC_CHUNK
        for g in range(n_groups):
          b = lane + g * LANES
          d = plsc.load_gather(idx_v, [b * f_total + f])
          off = ((d >> 3) * (8 * B_TOTAL) + (b >> 7) * 1024
                 + (d & 7) * 128 + (b & 127))
          ok = (d < D_SC) & (off >= lo) & (off < lo + SC_CHUNK)
          plsc.store_scatter(buf, [off - lo], val16, mask=ok)

      def chunk_body(c, _):
        scatter_chunk(c, ones16)
        pltpu.sync_copy(buf,
                        out_hbm.at[pl.ds(f * PLANE + c * SC_CHUNK, SC_CHUNK)])
        scatter_chunk(c, zeros16)
        return 0

      lax.fori_loop(0, n_chunks, chunk_body, 0)

  return k(idx_flat)


def _tc_compute_body(f_total, sc_blocks, idxt_ref, out_ref):
  i = pl.program_id(0)
  d = lax.broadcasted_iota(jnp.int32, (f_total, D_BLK, B_TOTAL), 1)
  d = d + (i + sc_blocks) * D_BLK
  idx_b = idxt_ref[...][:, None, :]
  out_ref[...] = jnp.where(idx_b == d, jnp.float32(ON_VALUE),
                           jnp.float32(OFF_VALUE))


def _tc_assemble_body(f_total, img_ref, part_ref, out_ref, scratch, sem):
  del part_ref
  i = pl.program_id(0)
  cps = []
  for f in range(f_total):
    cps.append(pltpu.make_async_copy(
        img_ref.at[pl.ds(f * PLANE + i * SEG, SEG)], scratch.at[f], sem))
    cps[-1].start()
  for cp in cps:
    cp.wait()
  for f in range(f_total):
    for trl in range(D_BLK // 8):
      for c0 in range(B_TOTAL // 128):
        vals = scratch[f, pl.ds(trl * (8 * B_TOTAL) + c0 * 1024, 1024)]
        out_ref[f, pl.ds(8 * trl, 8), pl.ds(128 * c0, 128)] = (
            vals.reshape(8, 128))


@jax.jit
def kernel(indices):
  b_total, f_total = indices.shape
  sc_blocks = D_SC // D_BLK
  img = _one_hot_sc_image(indices.reshape(-1), f_total)
  idx_t = indices.T

  part = pl.pallas_call(
      functools.partial(_tc_compute_body, f_total, sc_blocks),
      grid=(DEPTH // D_BLK - sc_blocks,),
      in_specs=[pl.BlockSpec((f_total, b_total), lambda i: (0, 0))],
      out_specs=pl.BlockSpec((f_total, D_BLK, b_total),
                             lambda i: (0, i + sc_blocks, 0)),
      out_shape=jax.ShapeDtypeStruct((f_total, DEPTH, b_total), jnp.float32),
  )(idx_t)

  out_t = pl.pallas_call(
      functools.partial(_tc_assemble_body, f_total),
      grid=(sc_blocks,),
      in_specs=[
          pl.BlockSpec(memory_space=pl.ANY),
          pl.BlockSpec(memory_space=pl.ANY),
      ],
      out_specs=pl.BlockSpec((f_total, D_BLK, b_total), lambda i: (0, i, 0)),
      out_shape=jax.ShapeDtypeStruct((f_total, DEPTH, b_total), jnp.float32),
      scratch_shapes=[pltpu.VMEM((f_total, SEG), jnp.float32),
                      pltpu.SemaphoreType.DMA],
      input_output_aliases={1: 0},
  )(img, part)
  return jnp.transpose(out_t, (2, 0, 1))


# speedup vs baseline: 3.3413x; 1.1630x over previous
import functools

import jax
import jax.numpy as jnp
from jax import lax
from jax.experimental import pallas as pl
from jax.experimental.pallas import tpu as pltpu
from jax.experimental.pallas import tpu_sc as plsc

DEPTH = 1000
ON_VALUE = 1.0
OFF_VALUE = 0.0

NUM_CORES = 2
NUM_SUBCORES = 16
NUM_WORKERS = NUM_CORES * NUM_SUBCORES
LANES = 16

B_TOTAL = 1024
D_SC = 80
D_BLK = 40
PLANE = D_SC * B_TOTAL
SC_CHUNK = PLANE // 2
SEG = D_BLK * B_TOTAL


def _one_hot_sc_image(idx_flat, f_total):
  n_chunks = PLANE // SC_CHUNK
  n_groups = B_TOTAL // LANES

  mesh = plsc.VectorSubcoreMesh(core_axis_name="c", subcore_axis_name="s")

  @functools.partial(
      pl.kernel,
      mesh=mesh,
      out_type=jax.ShapeDtypeStruct((f_total * PLANE,), jnp.float32),
      scratch_types=[
          pltpu.VMEM((f_total * B_TOTAL,), jnp.int32),
          pltpu.VMEM((SC_CHUNK,), jnp.float32),
      ],
      compiler_params=pltpu.CompilerParams(needs_layout_passes=False),
  )
  def k(idx_hbm, out_hbm, idx_v, buf):
    wid = lax.axis_index("s") * NUM_CORES + lax.axis_index("c")

    @pl.when(wid < f_total)
    def _worker():
      f = wid
      pltpu.sync_copy(idx_hbm, idx_v)

      zeros16 = jnp.zeros((LANES,), jnp.float32)

      def zero_body(i, _):
        base = i * (8 * LANES)
        for u in range(8):
          buf[pl.ds(base + u * LANES, LANES)] = zeros16
        return 0

      lax.fori_loop(0, SC_CHUNK // (8 * LANES), zero_body, 0)

      lane = lax.iota(jnp.int32, LANES)
      ones16 = jnp.full((LANES,), jnp.float32(ON_VALUE))

      def scatter_chunk(c, val16):
        lo = c * SC_CHUNK
        for g in range(n_groups):
          b = lane + g * LANES
          d = plsc.load_gather(idx_v, [b * f_total + f])
          off = ((d >> 3) * (8 * B_TOTAL) + (b >> 7) * 1024
                 + (d & 7) * 128 + (b & 127))
          ok = (d < D_SC) & (off >= lo) & (off < lo + SC_CHUNK)
          plsc.store_scatter(buf, [off - lo], val16, mask=ok)

      def chunk_body(c, _):
        scatter_chunk(c, ones16)
        pltpu.sync_copy(buf,
                        out_hbm.at[pl.ds(f * PLANE + c * SC_CHUNK, SC_CHUNK)])
        scatter_chunk(c, zeros16)
        return 0

      lax.fori_loop(0, n_chunks, chunk_body, 0)

  return k(idx_flat)


def _tc_compute_body(f_total, sc_blocks, idxt_ref, out_ref):
  i = pl.program_id(0)
  d = lax.broadcasted_iota(jnp.int32, (f_total, D_BLK, B_TOTAL), 1)
  d = d + (i + sc_blocks) * D_BLK
  idx_b = idxt_ref[...][:, None, :]
  out_ref[...] = jnp.where(idx_b == d, jnp.float32(ON_VALUE),
                           jnp.float32(OFF_VALUE))


def _tc_assemble_body(f_total, img_ref, part_ref, out_ref, scratch, sem):
  del part_ref
  i = pl.program_id(0)
  cps = []
  for f in range(f_total):
    cps.append(pltpu.make_async_copy(
        img_ref.at[pl.ds(f * PLANE + i * SEG, SEG)], scratch.at[f], sem))
    cps[-1].start()
  for cp in cps:
    cp.wait()
  for f in range(f_total):
    for trl in range(D_BLK // 8):
      for c0 in range(B_TOTAL // 128):
        vals = scratch[f, pl.ds(trl * (8 * B_TOTAL) + c0 * 1024, 1024)]
        out_ref[f, pl.ds(8 * trl, 8), pl.ds(128 * c0, 128)] = (
            vals.reshape(8, 128))


@jax.jit
def kernel(indices):
  b_total, f_total = indices.shape
  sc_blocks = D_SC // D_BLK
  img = _one_hot_sc_image(indices.reshape(-1), f_total)
  idx_t = indices.T

  part = pl.pallas_call(
      functools.partial(_tc_compute_body, f_total, sc_blocks),
      grid=(DEPTH // D_BLK - sc_blocks,),
      in_specs=[pl.BlockSpec((f_total, b_total), lambda i: (0, 0))],
      out_specs=pl.BlockSpec((f_total, D_BLK, b_total),
                             lambda i: (0, i + sc_blocks, 0)),
      out_shape=jax.ShapeDtypeStruct((f_total, DEPTH, b_total), jnp.float32),
  )(idx_t)

  out_t = pl.pallas_call(
      functools.partial(_tc_assemble_body, f_total),
      grid=(sc_blocks,),
      in_specs=[
          pl.BlockSpec(memory_space=pl.ANY),
          pl.BlockSpec(memory_space=pl.ANY),
      ],
      out_specs=pl.BlockSpec((f_total, D_BLK, b_total), lambda i: (0, i, 0)),
      out_shape=jax.ShapeDtypeStruct((f_total, DEPTH, b_total), jnp.float32),
      scratch_shapes=[pltpu.VMEM((f_total, SEG), jnp.float32),
                      pltpu.SemaphoreType.DMA],
      input_output_aliases={1: 0},
  )(img, part)
  return jnp.transpose(out_t, (2, 0, 1))
